# baseline (speedup 1.0000x reference)
import functools

import jax
import jax.numpy as jnp
from jax import lax
from jax.experimental import pallas as pl
from jax.experimental.pallas import tpu as pltpu
from jax.experimental.pallas import tpu_sc as plsc

N = 10000
E = 320000
H = 128
NC = 2
NS = 16
NW = NC * NS
BK = 128
NBLK = 78
CH = 13
NCH = NBLK // CH
XTRA = E - NW * NBLK * BK
RPT = 624
RREM = N - NS * RPT
BN = 400

_mesh = plsc.VectorSubcoreMesh(
    core_axis_name="c", subcore_axis_name="s", num_cores=NC, num_subcores=NS
)


@functools.partial(
    pl.kernel,
    out_type=(
        jax.ShapeDtypeStruct((NC, N, H), jnp.float32),
        jax.ShapeDtypeStruct((NW, N), jnp.float32),
    ),
    mesh=_mesh,
    scratch_types=[
        pltpu.VMEM((CH * BK,), jnp.int32),
        pltpu.VMEM((CH * BK,), jnp.int32),
        pltpu.VMEM((CH * BK,), jnp.int32),
        pltpu.VMEM((CH * BK,), jnp.int32),
        pltpu.VMEM((BK,), jnp.int32),
        pltpu.VMEM((BK,), jnp.int32),
        pltpu.VMEM((BK, H), jnp.float32),
        pltpu.VMEM((BK, H), jnp.float32),
        pltpu.VMEM((N,), jnp.float32),
        pltpu.VMEM_SHARED((N, H), jnp.float32),
        pltpu.SemaphoreType.DMA,
        pltpu.SemaphoreType.DMA,
        pltpu.SemaphoreType.DMA,
        pltpu.SemaphoreType.DMA,
    ],
    compiler_params=pltpu.CompilerParams(needs_layout_passes=False),
)
def _sc_aggregate(h_hbm, srcx_hbm, dstx_hbm, zrows_hbm, zflat_hbm,
                  agg_hbm, deg_hbm,
                  isbuf0, isbuf1, idbuf0, idbuf1, xsbuf, xdbuf,
                  gbuf0, gbuf1, deg_v, agg_sh,
                  isem0, isem1, sem0, sem1):
    c = lax.axis_index("c")
    s = lax.axis_index("s")
    wid = s * NC + c

    pltpu.sync_copy(zrows_hbm, agg_sh.at[pl.ds(s * RPT, RPT)])

    @pl.when(s == NS - 1)
    def _():
        pltpu.sync_copy(zrows_hbm.at[pl.ds(0, RREM)],
                        agg_sh.at[pl.ds(NS * RPT, RREM)])

    pltpu.sync_copy(zflat_hbm, deg_v)
    plsc.subcore_barrier()

    ones = jnp.ones((16,), jnp.float32)
    isbufs = (isbuf0, isbuf1)
    idbufs = (idbuf0, idbuf1)
    isems = (isem0, isem1)
    gbufs = (gbuf0, gbuf1)
    gsems = (sem0, sem1)

    ecnt = NBLK * BK
    ck = CH * BK

    def _stage(k, p):
        base = wid * ecnt + k * ck
        pltpu.async_copy(srcx_hbm.at[pl.ds(base, ck)], isbufs[p], isems[p])
        pltpu.async_copy(dstx_hbm.at[pl.ds(base, ck)], idbufs[p], isems[p])

    def _stage_wait(k, p):
        base = wid * ecnt + k * ck
        pltpu.make_async_copy(
            srcx_hbm.at[pl.ds(base, ck)], isbufs[p], isems[p]).wait()
        pltpu.make_async_copy(
            dstx_hbm.at[pl.ds(base, ck)], idbufs[p], isems[p]).wait()

    def _gather(ib, jj, b):
        return pltpu.async_copy(
            h_hbm.at[ib.at[pl.ds(jj * BK, BK)]], gbufs[b], gsems[b])

    def _deg_block(idx_ref, off):
        for g in range(BK // 16):
            v = idx_ref[pl.ds(off + g * 16, 16)]
            plsc.addupdate_scatter(deg_v, [v], ones)

    _stage(0, 0)
    _stage_wait(0, 0)
    _gather(isbuf0, 0, 0)

    def body(k2, carry):
        for p in range(2):
            k = 2 * k2 + p
            isb = isbufs[p]
            idb = idbufs[p]
            for jj in range(CH):
                b = (p + jj) % 2
                if jj == 1:
                    @pl.when(k + 1 < NCH)
                    def _():
                        _stage(k + 1, 1 - p)
                if jj < CH - 1:
                    _gather(isb, jj + 1, 1 - b)
                else:
                    @pl.when(k + 1 < NCH)
                    def _():
                        _stage_wait(k + 1, 1 - p)
                        _gather(isbufs[1 - p], 0, 1 - b)
                pltpu.make_async_copy(
                    h_hbm.at[isb.at[pl.ds(jj * BK, BK)]], gbufs[b],
                    gsems[b]).wait()
                pltpu.sync_copy(
                    gbufs[b], agg_sh.at[idb.at[pl.ds(jj * BK, BK)]], add=True)
                _deg_block(idb, jj * BK)
        return carry

    lax.fori_loop(0, NCH // 2, body, 0)

    @pl.when(wid < XTRA // BK)
    def _():
        base = NW * ecnt + wid * BK
        pltpu.sync_copy(srcx_hbm.at[pl.ds(base, BK)], xsbuf)
        pltpu.sync_copy(dstx_hbm.at[pl.ds(base, BK)], xdbuf)
        pltpu.async_copy(h_hbm.at[xsbuf], gbuf0, sem0).wait()
        pltpu.sync_copy(gbuf0, agg_sh.at[xdbuf], add=True)
        _deg_block(xdbuf, 0)

    plsc.subcore_barrier()

    pltpu.sync_copy(
        agg_sh.at[pl.ds(s * RPT, RPT)],
        agg_hbm.at[c, pl.ds(s * RPT, RPT)],
    )

    @pl.when(s == NS - 1)
    def _():
        pltpu.sync_copy(agg_sh.at[pl.ds(NS * RPT, RREM)],
                        agg_hbm.at[c, pl.ds(NS * RPT, RREM)])

    pltpu.sync_copy(deg_v, deg_hbm.at[wid])


def _dot(a, b):
    return jnp.dot(a, b, preferred_element_type=jnp.float32,
                   precision=lax.Precision.DEFAULT)


def _mlp_body(h_ref, a0_ref, a1_ref, deg_ref, w1a_ref, w1b_ref, b1_ref,
              w2_ref, b2_ref, o_ref):
    deg = jnp.sum(deg_ref[...], axis=1, keepdims=True)
    inv = 1.0 / jnp.maximum(deg, 1.0)
    agg = (a0_ref[0] + a1_ref[0]) * inv
    y = _dot(h_ref[...], w1a_ref[...]) + _dot(agg, w1b_ref[...]) + b1_ref[...]
    y = jnp.maximum(y, 0.0)
    z = _dot(y, w2_ref[...]) + b2_ref[...]
    o_ref[...] = jnp.maximum(z, 0.0)


_mlp = pl.pallas_call(
    _mlp_body,
    grid=(N // BN,),
    in_specs=[
        pl.BlockSpec((BN, H), lambda i: (i, 0)),
        pl.BlockSpec((1, BN, H), lambda i: (0, i, 0)),
        pl.BlockSpec((1, BN, H), lambda i: (1, i, 0)),
        pl.BlockSpec((BN, NW), lambda i: (i, 0)),
        pl.BlockSpec((H, H), lambda i: (0, 0)),
        pl.BlockSpec((H, H), lambda i: (0, 0)),
        pl.BlockSpec((1, H), lambda i: (0, 0)),
        pl.BlockSpec((H, H), lambda i: (0, 0)),
        pl.BlockSpec((1, H), lambda i: (0, 0)),
    ],
    out_specs=pl.BlockSpec((BN, H), lambda i: (i, 0)),
    out_shape=jax.ShapeDtypeStruct((N, H), jnp.float32),
)


def kernel(h, edge_index, W1, b1, W2, b2):
    src = edge_index[0].astype(jnp.int32)
    dst = edge_index[1].astype(jnp.int32)
    zrows = jnp.zeros((RPT, H), jnp.float32)
    zflat = jnp.zeros((N,), jnp.float32)

    agg_parts, deg_parts = _sc_aggregate(h, src, dst, zrows, zflat)

    return _mlp(h, agg_parts, agg_parts, deg_parts.T,
                W1[:H], W1[H:], b1.reshape(1, H), W2, b2.reshape(1, H))


# speedup vs baseline: 1.0700x; 1.0700x over previous
import functools

import jax
import jax.numpy as jnp
from jax import lax
from jax.experimental import pallas as pl
from jax.experimental.pallas import tpu as pltpu
from jax.experimental.pallas import tpu_sc as plsc

N = 10000
E = 320000
H = 128
NPAD = 10240
NC = 2
NS = 16
NW = NC * NS
BK = 125
NBLK = 80
CH = 8
NCH = NBLK // CH
ROWS_PER_TILE = NPAD // NS
BN = 400

_mesh = plsc.VectorSubcoreMesh(
    core_axis_name="c", subcore_axis_name="s", num_cores=NC, num_subcores=NS
)


@functools.partial(
    pl.kernel,
    out_type=(
        jax.ShapeDtypeStruct((NC, NPAD, H), jnp.float32),
        jax.ShapeDtypeStruct((NW, NPAD), jnp.float32),
    ),
    mesh=_mesh,
    scratch_types=[
        pltpu.VMEM((CH, BK), jnp.int32),
        pltpu.VMEM((CH, BK), jnp.int32),
        pltpu.VMEM((CH, BK), jnp.int32),
        pltpu.VMEM((CH, BK), jnp.int32),
        pltpu.VMEM((BK, H), jnp.float32),
        pltpu.VMEM((BK, H), jnp.float32),
        pltpu.VMEM((NPAD,), jnp.float32),
        pltpu.VMEM_SHARED((NPAD, H), jnp.float32),
        pltpu.SemaphoreType.DMA,
        pltpu.SemaphoreType.DMA,
        pltpu.SemaphoreType.DMA,
        pltpu.SemaphoreType.DMA,
    ],
    compiler_params=pltpu.CompilerParams(needs_layout_passes=False),
)
def _sc_aggregate(h_hbm, idx_hbm, zrows_hbm, zflat_hbm,
                  agg_hbm, deg_hbm,
                  isbuf0, isbuf1, idbuf0, idbuf1, gbuf0, gbuf1, deg_v, agg_sh,
                  isem0, isem1, sem0, sem1):
    c = lax.axis_index("c")
    s = lax.axis_index("s")
    wid = s * NC + c

    pltpu.sync_copy(zrows_hbm, agg_sh.at[pl.ds(s * ROWS_PER_TILE, ROWS_PER_TILE)])
    pltpu.sync_copy(zflat_hbm, deg_v)
    plsc.subcore_barrier()

    ones = jnp.ones((16,), jnp.float32)
    lane = lax.iota(jnp.int32, 16)
    tail_mask = lane >= 3
    isbufs = (isbuf0, isbuf1)
    idbufs = (idbuf0, idbuf1)
    isems = (isem0, isem1)
    gbufs = (gbuf0, gbuf1)
    gsems = (sem0, sem1)

    def _stage(k, p):
        pltpu.async_copy(idx_hbm.at[0, wid, pl.ds(k * CH, CH)], isbufs[p], isems[p])
        pltpu.async_copy(idx_hbm.at[1, wid, pl.ds(k * CH, CH)], idbufs[p], isems[p])

    def _stage_wait(k, p):
        pltpu.make_async_copy(
            idx_hbm.at[0, wid, pl.ds(k * CH, CH)], isbufs[p], isems[p]).wait()
        pltpu.make_async_copy(
            idx_hbm.at[1, wid, pl.ds(k * CH, CH)], idbufs[p], isems[p]).wait()

    def _gather(ib, jj, b):
        return pltpu.async_copy(h_hbm.at[ib.at[jj]], gbufs[b], gsems[b])

    _stage(0, 0)
    _stage_wait(0, 0)
    _gather(isbuf0, 0, 0)

    def body(k2, carry):
        for p in range(2):
            k = 2 * k2 + p
            isb = isbufs[p]
            idb = idbufs[p]
            for jj in range(CH):
                b = jj % 2
                if jj == 1:
                    @pl.when(k + 1 < NCH)
                    def _():
                        _stage(k + 1, 1 - p)
                if jj < CH - 1:
                    _gather(isb, jj + 1, 1 - b)
                else:
                    @pl.when(k + 1 < NCH)
                    def _():
                        _stage_wait(k + 1, 1 - p)
                        _gather(isbufs[1 - p], 0, 1 - b)
                pltpu.make_async_copy(
                    h_hbm.at[isb.at[jj]], gbufs[b], gsems[b]).wait()
                pltpu.sync_copy(gbufs[b], agg_sh.at[idb.at[jj]], add=True)
                for g in range(7):
                    v = idb[jj, pl.ds(g * 16, 16)]
                    plsc.addupdate_scatter(deg_v, [v], ones)
                v = idb[jj, pl.ds(BK - 16, 16)]
                plsc.addupdate_scatter(deg_v, [v], ones, mask=tail_mask)
        return carry

    lax.fori_loop(0, NCH // 2, body, 0)
    plsc.subcore_barrier()

    pltpu.sync_copy(
        agg_sh.at[pl.ds(s * ROWS_PER_TILE, ROWS_PER_TILE)],
        agg_hbm.at[c, pl.ds(s * ROWS_PER_TILE, ROWS_PER_TILE)],
    )
    pltpu.sync_copy(deg_v, deg_hbm.at[wid])


def _dot(a, b):
    return jnp.dot(a, b, preferred_element_type=jnp.float32,
                   precision=lax.Precision.DEFAULT)


def _mlp_body(h_ref, a0_ref, a1_ref, deg_ref, w1a_ref, w1b_ref, b1_ref,
              w2_ref, b2_ref, o_ref):
    deg = jnp.sum(deg_ref[...], axis=1, keepdims=True)
    inv = 1.0 / jnp.maximum(deg, 1.0)
    agg = (a0_ref[0] + a1_ref[0]) * inv
    y = _dot(h_ref[...], w1a_ref[...]) + _dot(agg, w1b_ref[...]) + b1_ref[...]
    y = jnp.maximum(y, 0.0)
    z = _dot(y, w2_ref[...]) + b2_ref[...]
    o_ref[...] = jnp.maximum(z, 0.0)


_mlp = pl.pallas_call(
    _mlp_body,
    grid=(N // BN,),
    in_specs=[
        pl.BlockSpec((BN, H), lambda i: (i, 0)),
        pl.BlockSpec((1, BN, H), lambda i: (0, i, 0)),
        pl.BlockSpec((1, BN, H), lambda i: (1, i, 0)),
        pl.BlockSpec((BN, NW), lambda i: (i, 0)),
        pl.BlockSpec((H, H), lambda i: (0, 0)),
        pl.BlockSpec((H, H), lambda i: (0, 0)),
        pl.BlockSpec((1, H), lambda i: (0, 0)),
        pl.BlockSpec((H, H), lambda i: (0, 0)),
        pl.BlockSpec((1, H), lambda i: (0, 0)),
    ],
    out_specs=pl.BlockSpec((BN, H), lambda i: (i, 0)),
    out_shape=jax.ShapeDtypeStruct((N, H), jnp.float32),
)


def kernel(h, edge_index, W1, b1, W2, b2):
    idx = edge_index.astype(jnp.int32).reshape(2, NW, NBLK, BK)
    zrows = jnp.zeros((ROWS_PER_TILE, H), jnp.float32)
    zflat = jnp.zeros((NPAD,), jnp.float32)

    agg_parts, deg_parts = _sc_aggregate(h, idx, zrows, zflat)

    return _mlp(h, agg_parts, agg_parts, deg_parts.T,
                W1[:H], W1[H:], b1.reshape(1, H), W2, b2.reshape(1, H))
